# Initial kernel scaffold; baseline (speedup 1.0000x reference)
import jax
import jax.numpy as jnp
from jax.experimental import pallas as pl


def kernel(x, edge_index, W0, b0, g0, be0, W1, b1, g1, be1, W2, b2, g2, be2, Wf, bf):
    raise NotImplementedError("write your pallas kernel here")


# speedup vs baseline: 5.7258x; 5.7258x over previous
import functools

import jax
import jax.numpy as jnp
from jax import lax
from jax.experimental import pallas as pl
from jax.experimental.pallas import tpu as pltpu
from jax.experimental.pallas import tpu_sc as plsc

N = 10000
D_IN = 128
H = 256
HALF = 128
E = 320000
EP = 327680
NSUB = 16
NE_TILE = EP // NSUB
CHUNK = 128
NCHUNK = NE_TILE // CHUNK
ROWS_TILE = 632
ROWS_LAST = N - 15 * ROWS_TILE
NACC = NSUB * ROWS_TILE

_mesh = plsc.VectorSubcoreMesh(core_axis_name="c", subcore_axis_name="s")


@functools.partial(
    pl.kernel,
    mesh=_mesh,
    out_type=jax.ShapeDtypeStruct((N, 16), jnp.float32),
    scratch_types=[
        pltpu.VMEM((CHUNK,), jnp.int32),
        pltpu.VMEM((CHUNK, 16), jnp.float32),
        pltpu.VMEM_SHARED((NACC, 16), jnp.float32),
    ],
)
def _deg_kernel(dst_hbm, ones_hbm, deg_out, dst_v, ones_v, acc_sh):
    c = lax.axis_index("c")
    s = lax.axis_index("s")

    @pl.when(c == 0)
    def _():
        pltpu.sync_copy(ones_hbm, ones_v)
        r0 = s * ROWS_TILE
        for j in range(4):
            pltpu.sync_copy(ones_v, acc_sh.at[pl.ds(r0 + j * CHUNK, CHUNK)])
        pltpu.sync_copy(ones_v.at[pl.ds(0, ROWS_TILE - 4 * CHUNK)],
                        acc_sh.at[pl.ds(r0 + 4 * CHUNK, ROWS_TILE - 4 * CHUNK)])
        plsc.subcore_barrier()

        base0 = s * NE_TILE

        def body(k, carry):
            pltpu.sync_copy(dst_hbm.at[pl.ds(base0 + k * CHUNK, CHUNK)], dst_v)
            pltpu.sync_copy(ones_v, acc_sh.at[dst_v], add=True)
            return carry

        lax.fori_loop(0, NCHUNK, body, 0)
        plsc.subcore_barrier()

        @pl.when(s < NSUB - 1)
        def _():
            pltpu.sync_copy(acc_sh.at[pl.ds(r0, ROWS_TILE)],
                            deg_out.at[pl.ds(r0, ROWS_TILE)])

        @pl.when(s == NSUB - 1)
        def _():
            pltpu.sync_copy(acc_sh.at[pl.ds(r0, ROWS_LAST)],
                            deg_out.at[pl.ds(r0, ROWS_LAST)])


@functools.partial(
    pl.kernel,
    mesh=_mesh,
    out_type=[jax.ShapeDtypeStruct((N, HALF), jnp.float32),
              jax.ShapeDtypeStruct((N, HALF), jnp.float32)],
    scratch_types=[
        pltpu.VMEM((CHUNK,), jnp.int32),
        pltpu.VMEM((CHUNK,), jnp.int32),
        pltpu.VMEM((CHUNK, HALF), jnp.float32),
        pltpu.VMEM_SHARED((NACC, HALF), jnp.float32),
        pltpu.SemaphoreType.DMA,
    ],
)
def _agg_kernel(hw_lo, hw_hi, src_hbm, dst_hbm, out_lo, out_hi,
                src_v, dst_v, rows_v, acc_sh, sem):
    c = lax.axis_index("c")
    s = lax.axis_index("s")

    def run_half(hw, out):
        r0 = s * ROWS_TILE

        @pl.when(s < NSUB - 1)
        def _():
            pltpu.sync_copy(hw.at[pl.ds(r0, ROWS_TILE)],
                            acc_sh.at[pl.ds(r0, ROWS_TILE)])

        @pl.when(s == NSUB - 1)
        def _():
            pltpu.sync_copy(hw.at[pl.ds(r0, ROWS_LAST)],
                            acc_sh.at[pl.ds(r0, ROWS_LAST)])

        plsc.subcore_barrier()
        base0 = s * NE_TILE

        def body(k, carry):
            base = base0 + k * CHUNK
            pltpu.sync_copy(src_hbm.at[pl.ds(base, CHUNK)], src_v)
            pltpu.sync_copy(dst_hbm.at[pl.ds(base, CHUNK)], dst_v)
            pltpu.async_copy(hw.at[src_v], rows_v, sem).wait()
            pltpu.sync_copy(rows_v, acc_sh.at[dst_v], add=True)
            return carry

        lax.fori_loop(0, NCHUNK, body, 0)
        plsc.subcore_barrier()

        @pl.when(s < NSUB - 1)
        def _():
            pltpu.sync_copy(acc_sh.at[pl.ds(r0, ROWS_TILE)],
                            out.at[pl.ds(r0, ROWS_TILE)])

        @pl.when(s == NSUB - 1)
        def _():
            pltpu.sync_copy(acc_sh.at[pl.ds(r0, ROWS_LAST)],
                            out.at[pl.ds(r0, ROWS_LAST)])

    @pl.when(c == 0)
    def _():
        run_half(hw_lo, out_lo)

    @pl.when(c == 1)
    def _():
        run_half(hw_hi, out_hi)


_BR = 2000


def _mm0_body(deg_ref, x_ref, w_ref, lo_ref, hi_ref, dinv_ref):
    dinv = lax.rsqrt(deg_ref[...][:, :1])
    hw = jnp.dot(x_ref[...], w_ref[...], preferred_element_type=jnp.float32)
    hw = hw * dinv
    lo_ref[...] = hw[:, :HALF]
    hi_ref[...] = hw[:, HALF:]
    dinv_ref[...] = dinv


def _mm0_call(deg, x, w0):
    return pl.pallas_call(
        _mm0_body,
        grid=(N // _BR,),
        in_specs=[
            pl.BlockSpec((_BR, 16), lambda i: (i, 0)),
            pl.BlockSpec((_BR, D_IN), lambda i: (i, 0)),
            pl.BlockSpec((D_IN, H), lambda i: (0, 0)),
        ],
        out_specs=[
            pl.BlockSpec((_BR, HALF), lambda i: (i, 0)),
            pl.BlockSpec((_BR, HALF), lambda i: (i, 0)),
            pl.BlockSpec((_BR, 1), lambda i: (i, 0)),
        ],
        out_shape=[
            jax.ShapeDtypeStruct((N, HALF), jnp.float32),
            jax.ShapeDtypeStruct((N, HALF), jnp.float32),
            jax.ShapeDtypeStruct((N, 1), jnp.float32),
        ],
    )(deg, x, w0)


def _merge_body(lo_ref, hi_ref, dinv_ref, z_ref, s1_ref, s2_ref):
    i = pl.program_id(0)
    z = jnp.concatenate([lo_ref[...], hi_ref[...]], axis=1) * dinv_ref[...]
    z_ref[...] = z

    @pl.when(i == 0)
    def _():
        s1_ref[...] = jnp.zeros_like(s1_ref)
        s2_ref[...] = jnp.zeros_like(s2_ref)

    s1_ref[...] += jnp.sum(z, axis=0, keepdims=True)
    s2_ref[...] += jnp.sum(z * z, axis=0, keepdims=True)


def _merge_call(lo, hi, dinv):
    return pl.pallas_call(
        _merge_body,
        grid=(N // _BR,),
        in_specs=[
            pl.BlockSpec((_BR, HALF), lambda i: (i, 0)),
            pl.BlockSpec((_BR, HALF), lambda i: (i, 0)),
            pl.BlockSpec((_BR, 1), lambda i: (i, 0)),
        ],
        out_specs=[
            pl.BlockSpec((_BR, H), lambda i: (i, 0)),
            pl.BlockSpec((1, H), lambda i: (0, 0)),
            pl.BlockSpec((1, H), lambda i: (0, 0)),
        ],
        out_shape=[
            jax.ShapeDtypeStruct((N, H), jnp.float32),
            jax.ShapeDtypeStruct((1, H), jnp.float32),
            jax.ShapeDtypeStruct((1, H), jnp.float32),
        ],
    )(lo, hi, dinv)


def _bn_relu(z, s1, s2, g, be):
    mu = s1 * (1.0 / N)
    var = s2 * (1.0 / N) - mu * mu
    h = (z - mu) * (lax.rsqrt(var + 1e-5) * g) + be
    return jnp.maximum(h, 0.0)


def _norm_mm_body(z_ref, s1_ref, s2_ref, g_ref, be_ref, w_ref, dinv_ref,
                  lo_ref, hi_ref):
    h = _bn_relu(z_ref[...], s1_ref[...], s2_ref[...], g_ref[...], be_ref[...])
    hw = jnp.dot(h, w_ref[...], preferred_element_type=jnp.float32)
    hw = hw * dinv_ref[...]
    lo_ref[...] = hw[:, :HALF]
    hi_ref[...] = hw[:, HALF:]


def _norm_mm_call(z, s1, s2, g, be, w, dinv):
    return pl.pallas_call(
        _norm_mm_body,
        grid=(N // _BR,),
        in_specs=[
            pl.BlockSpec((_BR, H), lambda i: (i, 0)),
            pl.BlockSpec((1, H), lambda i: (0, 0)),
            pl.BlockSpec((1, H), lambda i: (0, 0)),
            pl.BlockSpec((1, H), lambda i: (0, 0)),
            pl.BlockSpec((1, H), lambda i: (0, 0)),
            pl.BlockSpec((H, H), lambda i: (0, 0)),
            pl.BlockSpec((_BR, 1), lambda i: (i, 0)),
        ],
        out_specs=[
            pl.BlockSpec((_BR, HALF), lambda i: (i, 0)),
            pl.BlockSpec((_BR, HALF), lambda i: (i, 0)),
        ],
        out_shape=[
            jax.ShapeDtypeStruct((N, HALF), jnp.float32),
            jax.ShapeDtypeStruct((N, HALF), jnp.float32),
        ],
    )(z, s1, s2, g, be, w, dinv)


def _head_body(z_ref, s1_ref, s2_ref, g_ref, be_ref, wf_ref, bf_ref, y_ref):
    h = _bn_relu(z_ref[...], s1_ref[...], s2_ref[...], g_ref[...], be_ref[...])
    y_ref[...] = jnp.dot(h, wf_ref[...],
                         preferred_element_type=jnp.float32) + bf_ref[...]


def _head_call(z, s1, s2, g, be, wf, bf):
    return pl.pallas_call(
        _head_body,
        grid=(N // _BR,),
        in_specs=[
            pl.BlockSpec((_BR, H), lambda i: (i, 0)),
            pl.BlockSpec((1, H), lambda i: (0, 0)),
            pl.BlockSpec((1, H), lambda i: (0, 0)),
            pl.BlockSpec((1, H), lambda i: (0, 0)),
            pl.BlockSpec((1, H), lambda i: (0, 0)),
            pl.BlockSpec((H, 1), lambda i: (0, 0)),
            pl.BlockSpec((1, 1), lambda i: (0, 0)),
        ],
        out_specs=pl.BlockSpec((_BR, 1), lambda i: (i, 0)),
        out_shape=jax.ShapeDtypeStruct((N, 1), jnp.float32),
    )(z, s1, s2, g, be, wf, bf)


def kernel(x, edge_index, W0, b0, g0, be0, W1, b1, g1, be1, W2, b2, g2, be2,
           Wf, bf):
    del b0, b1, b2
    src = edge_index[0]
    dst = edge_index[1]
    pad = EP - E
    srcp = jnp.concatenate([src, jnp.zeros((pad,), jnp.int32)])
    dstp = jnp.concatenate([dst, jnp.full((pad,), N, jnp.int32)])
    ones16 = jnp.ones((CHUNK, 16), jnp.float32)

    deg = _deg_kernel(dstp, ones16)
    lo, hi, dinv = _mm0_call(deg, x, W0)

    g0r, be0r = g0.reshape(1, H), be0.reshape(1, H)
    g1r, be1r = g1.reshape(1, H), be1.reshape(1, H)
    g2r, be2r = g2.reshape(1, H), be2.reshape(1, H)

    acc_lo, acc_hi = _agg_kernel(lo, hi, srcp, dstp)
    z, s1, s2 = _merge_call(acc_lo, acc_hi, dinv)
    lo, hi = _norm_mm_call(z, s1, s2, g0r, be0r, W1, dinv)

    acc_lo, acc_hi = _agg_kernel(lo, hi, srcp, dstp)
    z, s1, s2 = _merge_call(acc_lo, acc_hi, dinv)
    lo, hi = _norm_mm_call(z, s1, s2, g1r, be1r, W2, dinv)

    acc_lo, acc_hi = _agg_kernel(lo, hi, srcp, dstp)
    z, s1, s2 = _merge_call(acc_lo, acc_hi, dinv)
    return _head_call(z, s1, s2, g2r, be2r, Wf, bf.reshape(1, 1))


# speedup vs baseline: 7.9294x; 1.3848x over previous
import functools

import jax
import jax.numpy as jnp
from jax import lax
from jax.experimental import pallas as pl
from jax.experimental.pallas import tpu as pltpu
from jax.experimental.pallas import tpu_sc as plsc

N = 10000
D_IN = 128
H = 256
HALF = 128
E = 320000
EP = 327680
NSUB = 16
NE_TILE = EP // NSUB
CHUNK = 128
NCHUNK = NE_TILE // CHUNK
ROWS_TILE = 632
ROWS_LAST = N - 15 * ROWS_TILE
NACC = NSUB * ROWS_TILE

_mesh = plsc.VectorSubcoreMesh(core_axis_name="c", subcore_axis_name="s")

NBUF = 5
HCHUNK = NCHUNK // 2


@functools.partial(
    pl.kernel,
    mesh=_mesh,
    out_type=jax.ShapeDtypeStruct((2, N, 16), jnp.float32),
    scratch_types=[
        pltpu.VMEM((HCHUNK, CHUNK), jnp.int32),
        pltpu.VMEM((CHUNK, 16), jnp.float32),
        pltpu.VMEM_SHARED((NACC, 16), jnp.float32),
        pltpu.SemaphoreType.DMA,
        pltpu.SemaphoreType.DMA,
        pltpu.SemaphoreType.DMA,
        pltpu.SemaphoreType.DMA,
    ],
)
def _deg_kernel(dst_hbm, ones_hbm, deg_out, dslab, ones_v, acc_sh,
                s0, s1, s2, s3):
    c = lax.axis_index("c")
    s = lax.axis_index("s")
    sems = (s0, s1, s2, s3)

    pltpu.sync_copy(ones_hbm, ones_v)
    r0 = s * ROWS_TILE
    for j in range(4):
        pltpu.sync_copy(ones_v, acc_sh.at[pl.ds(r0 + j * CHUNK, CHUNK)])
    pltpu.sync_copy(ones_v.at[pl.ds(0, ROWS_TILE - 4 * CHUNK)],
                    acc_sh.at[pl.ds(r0 + 4 * CHUNK, ROWS_TILE - 4 * CHUNK)])
    pltpu.sync_copy(dst_hbm.at[s, pl.ds(c * HCHUNK, HCHUNK)], dslab)
    plsc.subcore_barrier()

    def body(g, carry):
        handles = []
        for b in range(4):
            k = g * 4 + b
            handles.append(pltpu.async_copy(
                ones_v, acc_sh.at[dslab.at[k]], sems[b], add=True))
        for h in handles:
            h.wait()
        return carry

    lax.fori_loop(0, HCHUNK // 4, body, 0)
    plsc.subcore_barrier()

    @pl.when(s < NSUB - 1)
    def _():
        pltpu.sync_copy(acc_sh.at[pl.ds(r0, ROWS_TILE)],
                        deg_out.at[c, pl.ds(r0, ROWS_TILE)])

    @pl.when(s == NSUB - 1)
    def _():
        pltpu.sync_copy(acc_sh.at[pl.ds(r0, ROWS_LAST)],
                        deg_out.at[c, pl.ds(r0, ROWS_LAST)])


@functools.partial(
    pl.kernel,
    mesh=_mesh,
    out_type=[jax.ShapeDtypeStruct((N, HALF), jnp.float32),
              jax.ShapeDtypeStruct((N, HALF), jnp.float32)],
    scratch_types=[
        [pltpu.VMEM((2, CHUNK), jnp.int32)] * 2,
        [pltpu.VMEM((CHUNK, HALF), jnp.float32)] * 2,
        pltpu.VMEM_SHARED((NACC, HALF), jnp.float32),
        [pltpu.SemaphoreType.DMA] * 2,
        [pltpu.SemaphoreType.DMA] * 2,
        [pltpu.SemaphoreType.DMA] * 2,
    ],
)
def _agg_kernel(hw_lo, hw_hi, sd_hbm, out_lo, out_hi,
                idxb, rowb, acc_sh, isems, gsems, ssems):
    c = lax.axis_index("c")
    s = lax.axis_index("s")

    def idx_issue(k, b):
        return pltpu.async_copy(sd_hbm.at[s, k], idxb[b], isems[b])

    def idx_wait(b):
        pltpu.make_async_copy(sd_hbm.at[s, 0], idxb[b], isems[b]).wait()

    def gather_issue(hw, b):
        pltpu.async_copy(hw.at[idxb[b].at[0]], rowb[b], gsems[b])

    def gather_wait(hw, b):
        pltpu.make_async_copy(hw.at[idxb[b].at[0]], rowb[b], gsems[b]).wait()

    def scatter(b):
        return pltpu.async_copy(rowb[b], acc_sh.at[idxb[b].at[1]], ssems[b],
                                add=True)

    def run_half(hw, out, acc_sh):
        r0 = s * ROWS_TILE
        h_i0 = idx_issue(0, 0)
        idx_issue(1, 1)

        @pl.when(s < NSUB - 1)
        def _():
            pltpu.sync_copy(hw.at[pl.ds(r0, ROWS_TILE)],
                            acc_sh.at[pl.ds(r0, ROWS_TILE)])

        @pl.when(s == NSUB - 1)
        def _():
            pltpu.sync_copy(hw.at[pl.ds(r0, ROWS_LAST)],
                            acc_sh.at[pl.ds(r0, ROWS_LAST)])

        h_i0.wait()
        gather_issue(hw, 0)
        plsc.subcore_barrier()

        def body(g, carry):
            for b in range(2):
                k = g * 2 + b
                nb = 1 - b

                @pl.when(k + 1 < NCHUNK)
                def _():
                    idx_wait(nb)
                    gather_issue(hw, nb)

                gather_wait(hw, b)
                scatter(b).wait()

                @pl.when(k + 2 < NCHUNK)
                def _():
                    idx_issue(k + 2, b)
            return carry

        lax.fori_loop(0, NCHUNK // 2, body, 0)
        plsc.subcore_barrier()

        @pl.when(s < NSUB - 1)
        def _():
            pltpu.sync_copy(acc_sh.at[pl.ds(r0, ROWS_TILE)],
                            out.at[pl.ds(r0, ROWS_TILE)])

        @pl.when(s == NSUB - 1)
        def _():
            pltpu.sync_copy(acc_sh.at[pl.ds(r0, ROWS_LAST)],
                            out.at[pl.ds(r0, ROWS_LAST)])

    @pl.when(c == 0)
    def _():
        run_half(hw_lo, out_lo, acc_sh)

    @pl.when(c == 1)
    def _():
        run_half(hw_hi, out_hi, acc_sh)


_BR = 2000


def _mm0_body(deg_ref, x_ref, w_ref, lo_ref, hi_ref, dinv_ref):
    deg = deg_ref[0, :, :1] + deg_ref[1, :, :1] - 1.0
    dinv = lax.rsqrt(deg)
    hw = jnp.dot(x_ref[...], w_ref[...], preferred_element_type=jnp.float32)
    hw = hw * dinv
    lo_ref[...] = hw[:, :HALF]
    hi_ref[...] = hw[:, HALF:]
    dinv_ref[...] = dinv


def _mm0_call(deg, x, w0):
    return pl.pallas_call(
        _mm0_body,
        grid=(N // _BR,),
        in_specs=[
            pl.BlockSpec((2, _BR, 16), lambda i: (0, i, 0)),
            pl.BlockSpec((_BR, D_IN), lambda i: (i, 0)),
            pl.BlockSpec((D_IN, H), lambda i: (0, 0)),
        ],
        out_specs=[
            pl.BlockSpec((_BR, HALF), lambda i: (i, 0)),
            pl.BlockSpec((_BR, HALF), lambda i: (i, 0)),
            pl.BlockSpec((_BR, 1), lambda i: (i, 0)),
        ],
        out_shape=[
            jax.ShapeDtypeStruct((N, HALF), jnp.float32),
            jax.ShapeDtypeStruct((N, HALF), jnp.float32),
            jax.ShapeDtypeStruct((N, 1), jnp.float32),
        ],
    )(deg, x, w0)


def _merge_body(lo_ref, hi_ref, dinv_ref, z_ref, s1_ref, s2_ref):
    i = pl.program_id(0)
    z = jnp.concatenate([lo_ref[...], hi_ref[...]], axis=1) * dinv_ref[...]
    z_ref[...] = z

    @pl.when(i == 0)
    def _():
        s1_ref[...] = jnp.zeros_like(s1_ref)
        s2_ref[...] = jnp.zeros_like(s2_ref)

    s1_ref[...] += jnp.sum(z, axis=0, keepdims=True)
    s2_ref[...] += jnp.sum(z * z, axis=0, keepdims=True)


def _merge_call(lo, hi, dinv):
    return pl.pallas_call(
        _merge_body,
        grid=(N // _BR,),
        in_specs=[
            pl.BlockSpec((_BR, HALF), lambda i: (i, 0)),
            pl.BlockSpec((_BR, HALF), lambda i: (i, 0)),
            pl.BlockSpec((_BR, 1), lambda i: (i, 0)),
        ],
        out_specs=[
            pl.BlockSpec((_BR, H), lambda i: (i, 0)),
            pl.BlockSpec((1, H), lambda i: (0, 0)),
            pl.BlockSpec((1, H), lambda i: (0, 0)),
        ],
        out_shape=[
            jax.ShapeDtypeStruct((N, H), jnp.float32),
            jax.ShapeDtypeStruct((1, H), jnp.float32),
            jax.ShapeDtypeStruct((1, H), jnp.float32),
        ],
    )(lo, hi, dinv)


def _bn_relu(z, s1, s2, g, be):
    mu = s1 * (1.0 / N)
    var = s2 * (1.0 / N) - mu * mu
    h = (z - mu) * (lax.rsqrt(var + 1e-5) * g) + be
    return jnp.maximum(h, 0.0)


def _norm_mm_body(z_ref, s1_ref, s2_ref, g_ref, be_ref, w_ref, dinv_ref,
                  lo_ref, hi_ref):
    h = _bn_relu(z_ref[...], s1_ref[...], s2_ref[...], g_ref[...], be_ref[...])
    hw = jnp.dot(h, w_ref[...], preferred_element_type=jnp.float32)
    hw = hw * dinv_ref[...]
    lo_ref[...] = hw[:, :HALF]
    hi_ref[...] = hw[:, HALF:]


def _norm_mm_call(z, s1, s2, g, be, w, dinv):
    return pl.pallas_call(
        _norm_mm_body,
        grid=(N // _BR,),
        in_specs=[
            pl.BlockSpec((_BR, H), lambda i: (i, 0)),
            pl.BlockSpec((1, H), lambda i: (0, 0)),
            pl.BlockSpec((1, H), lambda i: (0, 0)),
            pl.BlockSpec((1, H), lambda i: (0, 0)),
            pl.BlockSpec((1, H), lambda i: (0, 0)),
            pl.BlockSpec((H, H), lambda i: (0, 0)),
            pl.BlockSpec((_BR, 1), lambda i: (i, 0)),
        ],
        out_specs=[
            pl.BlockSpec((_BR, HALF), lambda i: (i, 0)),
            pl.BlockSpec((_BR, HALF), lambda i: (i, 0)),
        ],
        out_shape=[
            jax.ShapeDtypeStruct((N, HALF), jnp.float32),
            jax.ShapeDtypeStruct((N, HALF), jnp.float32),
        ],
    )(z, s1, s2, g, be, w, dinv)


def _head_body(z_ref, s1_ref, s2_ref, g_ref, be_ref, wf_ref, bf_ref, y_ref):
    h = _bn_relu(z_ref[...], s1_ref[...], s2_ref[...], g_ref[...], be_ref[...])
    y_ref[...] = jnp.dot(h, wf_ref[...],
                         preferred_element_type=jnp.float32) + bf_ref[...]


def _head_call(z, s1, s2, g, be, wf, bf):
    return pl.pallas_call(
        _head_body,
        grid=(N // _BR,),
        in_specs=[
            pl.BlockSpec((_BR, H), lambda i: (i, 0)),
            pl.BlockSpec((1, H), lambda i: (0, 0)),
            pl.BlockSpec((1, H), lambda i: (0, 0)),
            pl.BlockSpec((1, H), lambda i: (0, 0)),
            pl.BlockSpec((1, H), lambda i: (0, 0)),
            pl.BlockSpec((H, 1), lambda i: (0, 0)),
            pl.BlockSpec((1, 1), lambda i: (0, 0)),
        ],
        out_specs=pl.BlockSpec((_BR, 1), lambda i: (i, 0)),
        out_shape=jax.ShapeDtypeStruct((N, 1), jnp.float32),
    )(z, s1, s2, g, be, wf, bf)


def kernel(x, edge_index, W0, b0, g0, be0, W1, b1, g1, be1, W2, b2, g2, be2,
           Wf, bf):
    del b0, b1, b2
    src = edge_index[0]
    dst = edge_index[1]
    pad = EP - E
    srcp = jnp.concatenate([src, jnp.zeros((pad,), jnp.int32)])
    dstp = jnp.concatenate([dst, jnp.full((pad,), N, jnp.int32)])
    src3 = srcp.reshape(NSUB, NCHUNK, CHUNK)
    dst3 = dstp.reshape(NSUB, NCHUNK, CHUNK)
    sd4 = jnp.stack([src3, dst3], axis=2)
    ones16 = jnp.ones((CHUNK, 16), jnp.float32)

    deg2 = _deg_kernel(dst3, ones16)
    lo, hi, dinv = _mm0_call(deg2, x, W0)

    g0r, be0r = g0.reshape(1, H), be0.reshape(1, H)
    g1r, be1r = g1.reshape(1, H), be1.reshape(1, H)
    g2r, be2r = g2.reshape(1, H), be2.reshape(1, H)

    acc_lo, acc_hi = _agg_kernel(lo, hi, sd4)
    z, s1, s2 = _merge_call(acc_lo, acc_hi, dinv)
    lo, hi = _norm_mm_call(z, s1, s2, g0r, be0r, W1, dinv)

    acc_lo, acc_hi = _agg_kernel(lo, hi, sd4)
    z, s1, s2 = _merge_call(acc_lo, acc_hi, dinv)
    lo, hi = _norm_mm_call(z, s1, s2, g1r, be1r, W2, dinv)

    acc_lo, acc_hi = _agg_kernel(lo, hi, sd4)
    z, s1, s2 = _merge_call(acc_lo, acc_hi, dinv)
    return _head_call(z, s1, s2, g2r, be2r, Wf, bf.reshape(1, 1))


# speedup vs baseline: 7.9313x; 1.0003x over previous
import functools

import jax
import jax.numpy as jnp
from jax import lax
from jax.experimental import pallas as pl
from jax.experimental.pallas import tpu as pltpu
from jax.experimental.pallas import tpu_sc as plsc

N = 10000
D_IN = 128
H = 256
HALF = 128
E = 320000
EP = 327680
NSUB = 16
NE_TILE = EP // NSUB
CHUNK = 128
NCHUNK = NE_TILE // CHUNK
ROWS_TILE = 632
ROWS_LAST = N - 15 * ROWS_TILE
NACC = NSUB * ROWS_TILE

_mesh = plsc.VectorSubcoreMesh(core_axis_name="c", subcore_axis_name="s")

NBUF = 5
HCHUNK = NCHUNK // 2


@functools.partial(
    pl.kernel,
    mesh=_mesh,
    out_type=jax.ShapeDtypeStruct((2, N, 16), jnp.float32),
    scratch_types=[
        pltpu.VMEM((HCHUNK, CHUNK), jnp.int32),
        pltpu.VMEM((CHUNK, 16), jnp.float32),
        pltpu.VMEM_SHARED((NACC, 16), jnp.float32),
        pltpu.SemaphoreType.DMA,
        pltpu.SemaphoreType.DMA,
        pltpu.SemaphoreType.DMA,
        pltpu.SemaphoreType.DMA,
    ],
)
def _deg_kernel(dst_hbm, ones_hbm, deg_out, dslab, ones_v, acc_sh,
                s0, s1, s2, s3):
    c = lax.axis_index("c")
    s = lax.axis_index("s")
    sems = (s0, s1, s2, s3)

    pltpu.sync_copy(ones_hbm, ones_v)
    r0 = s * ROWS_TILE
    for j in range(4):
        pltpu.sync_copy(ones_v, acc_sh.at[pl.ds(r0 + j * CHUNK, CHUNK)])
    pltpu.sync_copy(ones_v.at[pl.ds(0, ROWS_TILE - 4 * CHUNK)],
                    acc_sh.at[pl.ds(r0 + 4 * CHUNK, ROWS_TILE - 4 * CHUNK)])
    pltpu.sync_copy(dst_hbm.at[s, pl.ds(c * HCHUNK, HCHUNK)], dslab)
    plsc.subcore_barrier()

    def body(g, carry):
        handles = []
        for b in range(4):
            k = g * 4 + b
            handles.append(pltpu.async_copy(
                ones_v, acc_sh.at[dslab.at[k]], sems[b], add=True))
        for h in handles:
            h.wait()
        return carry

    lax.fori_loop(0, HCHUNK // 4, body, 0)
    plsc.subcore_barrier()

    @pl.when(s < NSUB - 1)
    def _():
        pltpu.sync_copy(acc_sh.at[pl.ds(r0, ROWS_TILE)],
                        deg_out.at[c, pl.ds(r0, ROWS_TILE)])

    @pl.when(s == NSUB - 1)
    def _():
        pltpu.sync_copy(acc_sh.at[pl.ds(r0, ROWS_LAST)],
                        deg_out.at[c, pl.ds(r0, ROWS_LAST)])


@functools.partial(
    pl.kernel,
    mesh=_mesh,
    out_type=[jax.ShapeDtypeStruct((N, HALF), jnp.float32),
              jax.ShapeDtypeStruct((N, HALF), jnp.float32)],
    scratch_types=[
        [pltpu.VMEM((2, CHUNK), jnp.int32)] * 3,
        [pltpu.VMEM((CHUNK, HALF), jnp.float32)] * 3,
        pltpu.VMEM_SHARED((NACC, HALF), jnp.float32),
        [pltpu.SemaphoreType.DMA] * 3,
        [pltpu.SemaphoreType.DMA] * 3,
        [pltpu.SemaphoreType.DMA] * 3,
    ],
)
def _agg_kernel(hw_lo, hw_hi, sd_hbm, out_lo, out_hi,
                idxb, rowb, acc_sh, isems, gsems, ssems):
    c = lax.axis_index("c")
    s = lax.axis_index("s")

    def idx_issue(k, b):
        return pltpu.async_copy(sd_hbm.at[s, k], idxb[b], isems[b])

    def idx_wait(b):
        pltpu.make_async_copy(sd_hbm.at[s, 0], idxb[b], isems[b]).wait()

    def gather_issue(hw, b):
        pltpu.async_copy(hw.at[idxb[b].at[0]], rowb[b], gsems[b])

    def gather_wait(hw, b):
        pltpu.make_async_copy(hw.at[idxb[b].at[0]], rowb[b], gsems[b]).wait()

    def scatter(b):
        return pltpu.async_copy(rowb[b], acc_sh.at[idxb[b].at[1]], ssems[b],
                                add=True)

    def run_half(hw, out, acc_sh):
        r0 = s * ROWS_TILE
        h_i0 = idx_issue(0, 0)
        h_i1 = idx_issue(1, 1)
        idx_issue(2, 2)

        @pl.when(s < NSUB - 1)
        def _():
            pltpu.sync_copy(hw.at[pl.ds(r0, ROWS_TILE)],
                            acc_sh.at[pl.ds(r0, ROWS_TILE)])

        @pl.when(s == NSUB - 1)
        def _():
            pltpu.sync_copy(hw.at[pl.ds(r0, ROWS_LAST)],
                            acc_sh.at[pl.ds(r0, ROWS_LAST)])

        h_i0.wait()
        gather_issue(hw, 0)
        h_i1.wait()
        gather_issue(hw, 1)
        plsc.subcore_barrier()

        def body(g, carry):
            for b in range(3):
                k = g * 3 + b
                nb = (b + 2) % 3

                @pl.when(k + 2 < NCHUNK)
                def _():
                    idx_wait(nb)
                    gather_issue(hw, nb)

                @pl.when(k < NCHUNK)
                def _():
                    gather_wait(hw, b)
                    scatter(b).wait()

                @pl.when(k + 3 < NCHUNK)
                def _():
                    idx_issue(k + 3, b)
            return carry

        lax.fori_loop(0, NCHUNK // 3 + 1, body, 0)
        plsc.subcore_barrier()

        @pl.when(s < NSUB - 1)
        def _():
            pltpu.sync_copy(acc_sh.at[pl.ds(r0, ROWS_TILE)],
                            out.at[pl.ds(r0, ROWS_TILE)])

        @pl.when(s == NSUB - 1)
        def _():
            pltpu.sync_copy(acc_sh.at[pl.ds(r0, ROWS_LAST)],
                            out.at[pl.ds(r0, ROWS_LAST)])

    @pl.when(c == 0)
    def _():
        run_half(hw_lo, out_lo, acc_sh)

    @pl.when(c == 1)
    def _():
        run_half(hw_hi, out_hi, acc_sh)


_BR = 2000


def _mm0_body(deg_ref, x_ref, w_ref, lo_ref, hi_ref, dinv_ref):
    deg = deg_ref[0, :, :1] + deg_ref[1, :, :1] - 1.0
    dinv = lax.rsqrt(deg)
    hw = jnp.dot(x_ref[...], w_ref[...], preferred_element_type=jnp.float32)
    hw = hw * dinv
    lo_ref[...] = hw[:, :HALF]
    hi_ref[...] = hw[:, HALF:]
    dinv_ref[...] = dinv


def _mm0_call(deg, x, w0):
    return pl.pallas_call(
        _mm0_body,
        grid=(N // _BR,),
        in_specs=[
            pl.BlockSpec((2, _BR, 16), lambda i: (0, i, 0)),
            pl.BlockSpec((_BR, D_IN), lambda i: (i, 0)),
            pl.BlockSpec((D_IN, H), lambda i: (0, 0)),
        ],
        out_specs=[
            pl.BlockSpec((_BR, HALF), lambda i: (i, 0)),
            pl.BlockSpec((_BR, HALF), lambda i: (i, 0)),
            pl.BlockSpec((_BR, 1), lambda i: (i, 0)),
        ],
        out_shape=[
            jax.ShapeDtypeStruct((N, HALF), jnp.float32),
            jax.ShapeDtypeStruct((N, HALF), jnp.float32),
            jax.ShapeDtypeStruct((N, 1), jnp.float32),
        ],
    )(deg, x, w0)


def _merge_body(lo_ref, hi_ref, dinv_ref, z_ref, s1_ref, s2_ref):
    i = pl.program_id(0)
    z = jnp.concatenate([lo_ref[...], hi_ref[...]], axis=1) * dinv_ref[...]
    z_ref[...] = z

    @pl.when(i == 0)
    def _():
        s1_ref[...] = jnp.zeros_like(s1_ref)
        s2_ref[...] = jnp.zeros_like(s2_ref)

    s1_ref[...] += jnp.sum(z, axis=0, keepdims=True)
    s2_ref[...] += jnp.sum(z * z, axis=0, keepdims=True)


def _merge_call(lo, hi, dinv):
    return pl.pallas_call(
        _merge_body,
        grid=(N // _BR,),
        in_specs=[
            pl.BlockSpec((_BR, HALF), lambda i: (i, 0)),
            pl.BlockSpec((_BR, HALF), lambda i: (i, 0)),
            pl.BlockSpec((_BR, 1), lambda i: (i, 0)),
        ],
        out_specs=[
            pl.BlockSpec((_BR, H), lambda i: (i, 0)),
            pl.BlockSpec((1, H), lambda i: (0, 0)),
            pl.BlockSpec((1, H), lambda i: (0, 0)),
        ],
        out_shape=[
            jax.ShapeDtypeStruct((N, H), jnp.float32),
            jax.ShapeDtypeStruct((1, H), jnp.float32),
            jax.ShapeDtypeStruct((1, H), jnp.float32),
        ],
    )(lo, hi, dinv)


def _bn_relu(z, s1, s2, g, be):
    mu = s1 * (1.0 / N)
    var = s2 * (1.0 / N) - mu * mu
    h = (z - mu) * (lax.rsqrt(var + 1e-5) * g) + be
    return jnp.maximum(h, 0.0)


def _norm_mm_body(z_ref, s1_ref, s2_ref, g_ref, be_ref, w_ref, dinv_ref,
                  lo_ref, hi_ref):
    h = _bn_relu(z_ref[...], s1_ref[...], s2_ref[...], g_ref[...], be_ref[...])
    hw = jnp.dot(h, w_ref[...], preferred_element_type=jnp.float32)
    hw = hw * dinv_ref[...]
    lo_ref[...] = hw[:, :HALF]
    hi_ref[...] = hw[:, HALF:]


def _norm_mm_call(z, s1, s2, g, be, w, dinv):
    return pl.pallas_call(
        _norm_mm_body,
        grid=(N // _BR,),
        in_specs=[
            pl.BlockSpec((_BR, H), lambda i: (i, 0)),
            pl.BlockSpec((1, H), lambda i: (0, 0)),
            pl.BlockSpec((1, H), lambda i: (0, 0)),
            pl.BlockSpec((1, H), lambda i: (0, 0)),
            pl.BlockSpec((1, H), lambda i: (0, 0)),
            pl.BlockSpec((H, H), lambda i: (0, 0)),
            pl.BlockSpec((_BR, 1), lambda i: (i, 0)),
        ],
        out_specs=[
            pl.BlockSpec((_BR, HALF), lambda i: (i, 0)),
            pl.BlockSpec((_BR, HALF), lambda i: (i, 0)),
        ],
        out_shape=[
            jax.ShapeDtypeStruct((N, HALF), jnp.float32),
            jax.ShapeDtypeStruct((N, HALF), jnp.float32),
        ],
    )(z, s1, s2, g, be, w, dinv)


def _head_body(z_ref, s1_ref, s2_ref, g_ref, be_ref, wf_ref, bf_ref, y_ref):
    h = _bn_relu(z_ref[...], s1_ref[...], s2_ref[...], g_ref[...], be_ref[...])
    y_ref[...] = jnp.dot(h, wf_ref[...],
                         preferred_element_type=jnp.float32) + bf_ref[...]


def _head_call(z, s1, s2, g, be, wf, bf):
    return pl.pallas_call(
        _head_body,
        grid=(N // _BR,),
        in_specs=[
            pl.BlockSpec((_BR, H), lambda i: (i, 0)),
            pl.BlockSpec((1, H), lambda i: (0, 0)),
            pl.BlockSpec((1, H), lambda i: (0, 0)),
            pl.BlockSpec((1, H), lambda i: (0, 0)),
            pl.BlockSpec((1, H), lambda i: (0, 0)),
            pl.BlockSpec((H, 1), lambda i: (0, 0)),
            pl.BlockSpec((1, 1), lambda i: (0, 0)),
        ],
        out_specs=pl.BlockSpec((_BR, 1), lambda i: (i, 0)),
        out_shape=jax.ShapeDtypeStruct((N, 1), jnp.float32),
    )(z, s1, s2, g, be, wf, bf)


def kernel(x, edge_index, W0, b0, g0, be0, W1, b1, g1, be1, W2, b2, g2, be2,
           Wf, bf):
    del b0, b1, b2
    src = edge_index[0]
    dst = edge_index[1]
    pad = EP - E
    srcp = jnp.concatenate([src, jnp.zeros((pad,), jnp.int32)])
    dstp = jnp.concatenate([dst, jnp.full((pad,), N, jnp.int32)])
    src3 = srcp.reshape(NSUB, NCHUNK, CHUNK)
    dst3 = dstp.reshape(NSUB, NCHUNK, CHUNK)
    sd4 = jnp.stack([src3, dst3], axis=2)
    ones16 = jnp.ones((CHUNK, 16), jnp.float32)

    deg2 = _deg_kernel(dst3, ones16)
    lo, hi, dinv = _mm0_call(deg2, x, W0)

    g0r, be0r = g0.reshape(1, H), be0.reshape(1, H)
    g1r, be1r = g1.reshape(1, H), be1.reshape(1, H)
    g2r, be2r = g2.reshape(1, H), be2.reshape(1, H)

    acc_lo, acc_hi = _agg_kernel(lo, hi, sd4)
    z, s1, s2 = _merge_call(acc_lo, acc_hi, dinv)
    lo, hi = _norm_mm_call(z, s1, s2, g0r, be0r, W1, dinv)

    acc_lo, acc_hi = _agg_kernel(lo, hi, sd4)
    z, s1, s2 = _merge_call(acc_lo, acc_hi, dinv)
    lo, hi = _norm_mm_call(z, s1, s2, g1r, be1r, W2, dinv)

    acc_lo, acc_hi = _agg_kernel(lo, hi, sd4)
    z, s1, s2 = _merge_call(acc_lo, acc_hi, dinv)
    return _head_call(z, s1, s2, g2r, be2r, Wf, bf.reshape(1, 1))


# speedup vs baseline: 10.5006x; 1.3239x over previous
import functools

import jax
import jax.numpy as jnp
from jax import lax
from jax.experimental import pallas as pl
from jax.experimental.pallas import tpu as pltpu
from jax.experimental.pallas import tpu_sc as plsc

N = 10000
D_IN = 128
H = 256
QW = 64
E = 320000
EP = 327680
NSUB = 16
NE_TILE = EP // NSUB
CHUNK = 128
NCHUNK = NE_TILE // CHUNK
ROWS_TILE = 632
ROWS_LAST = N - 15 * ROWS_TILE
NACC = NSUB * ROWS_TILE
NBUF = 5
HCHUNK = NCHUNK // 2

_mesh = plsc.VectorSubcoreMesh(core_axis_name="c", subcore_axis_name="s")


@functools.partial(
    pl.kernel,
    mesh=_mesh,
    out_type=jax.ShapeDtypeStruct((2, N, 16), jnp.float32),
    scratch_types=[
        pltpu.VMEM((HCHUNK, CHUNK), jnp.int32),
        pltpu.VMEM((CHUNK, 16), jnp.float32),
        pltpu.VMEM_SHARED((NACC, 16), jnp.float32),
        pltpu.SemaphoreType.DMA,
        pltpu.SemaphoreType.DMA,
        pltpu.SemaphoreType.DMA,
        pltpu.SemaphoreType.DMA,
    ],
)
def _deg_kernel(dst_hbm, ones_hbm, deg_out, dslab, ones_v, acc_sh,
                s0, s1, s2, s3):
    c = lax.axis_index("c")
    s = lax.axis_index("s")
    sems = (s0, s1, s2, s3)

    pltpu.sync_copy(ones_hbm, ones_v)
    r0 = s * ROWS_TILE
    for j in range(4):
        pltpu.sync_copy(ones_v, acc_sh.at[pl.ds(r0 + j * CHUNK, CHUNK)])
    pltpu.sync_copy(ones_v.at[pl.ds(0, ROWS_TILE - 4 * CHUNK)],
                    acc_sh.at[pl.ds(r0 + 4 * CHUNK, ROWS_TILE - 4 * CHUNK)])
    pltpu.sync_copy(dst_hbm.at[s, pl.ds(c * HCHUNK, HCHUNK)], dslab)
    plsc.subcore_barrier()

    def body(g, carry):
        handles = []
        for b in range(4):
            k = g * 4 + b
            handles.append(pltpu.async_copy(
                ones_v, acc_sh.at[dslab.at[k]], sems[b], add=True))
        for h in handles:
            h.wait()
        return carry

    lax.fori_loop(0, HCHUNK // 4, body, 0)
    plsc.subcore_barrier()

    @pl.when(s < NSUB - 1)
    def _():
        pltpu.sync_copy(acc_sh.at[pl.ds(r0, ROWS_TILE)],
                        deg_out.at[c, pl.ds(r0, ROWS_TILE)])

    @pl.when(s == NSUB - 1)
    def _():
        pltpu.sync_copy(acc_sh.at[pl.ds(r0, ROWS_LAST)],
                        deg_out.at[c, pl.ds(r0, ROWS_LAST)])


_QSD = jax.ShapeDtypeStruct((N, QW), jnp.float32)


@functools.partial(
    pl.kernel,
    mesh=_mesh,
    compiler_params=pltpu.CompilerParams(use_tc_tiling_on_sc=False),
    out_type=[_QSD, _QSD, _QSD, _QSD],
    scratch_types=[
        [pltpu.VMEM((2, CHUNK), jnp.int32)] * NBUF,
        [pltpu.VMEM((CHUNK, QW), jnp.float32)] * NBUF,
        pltpu.VMEM_SHARED((NACC, QW), jnp.float32),
        pltpu.VMEM_SHARED((NACC, QW), jnp.float32),
        [pltpu.SemaphoreType.DMA] * NBUF,
        [pltpu.SemaphoreType.DMA] * NBUF,
        [pltpu.SemaphoreType.DMA] * NBUF,
    ],
)
def _agg_kernel(h0, h1, h2, h3, sd_hbm, o0, o1, o2, o3,
                idxb, rowb, table_sh, acc_sh, isems, gsems, ssems):
    c = lax.axis_index("c")
    s = lax.axis_index("s")
    r0 = s * ROWS_TILE

    def idx_issue(k, b):
        return pltpu.async_copy(sd_hbm.at[s, k], idxb[b], isems[b])

    def idx_wait(b):
        pltpu.make_async_copy(sd_hbm.at[s, 0], idxb[b], isems[b]).wait()

    def gather_issue(b):
        pltpu.async_copy(table_sh.at[idxb[b].at[0]], rowb[b], gsems[b])

    def gather_wait(b):
        pltpu.make_async_copy(table_sh.at[idxb[b].at[0]], rowb[b],
                              gsems[b]).wait()

    def scatter(b):
        return pltpu.async_copy(rowb[b], acc_sh.at[idxb[b].at[1]], ssems[b],
                                add=True)

    def stage(h):
        @pl.when(s < NSUB - 1)
        def _():
            pltpu.sync_copy(h.at[pl.ds(r0, ROWS_TILE)],
                            table_sh.at[pl.ds(r0, ROWS_TILE)])
            pltpu.sync_copy(h.at[pl.ds(r0, ROWS_TILE)],
                            acc_sh.at[pl.ds(r0, ROWS_TILE)])

        @pl.when(s == NSUB - 1)
        def _():
            pltpu.sync_copy(h.at[pl.ds(r0, ROWS_LAST)],
                            table_sh.at[pl.ds(r0, ROWS_LAST)])
            pltpu.sync_copy(h.at[pl.ds(r0, ROWS_LAST)],
                            acc_sh.at[pl.ds(r0, ROWS_LAST)])

    def readback(out):
        @pl.when(s < NSUB - 1)
        def _():
            pltpu.sync_copy(acc_sh.at[pl.ds(r0, ROWS_TILE)],
                            out.at[pl.ds(r0, ROWS_TILE)])

        @pl.when(s == NSUB - 1)
        def _():
            pltpu.sync_copy(acc_sh.at[pl.ds(r0, ROWS_LAST)],
                            out.at[pl.ds(r0, ROWS_LAST)])

    def run_quarter():
        for b in range(NBUF):
            idx_issue(b, b)
        for b in range(NBUF - 1):
            idx_wait(b)
            gather_issue(b)

        def body(g, carry):
            for b in range(NBUF):
                k = g * NBUF + b
                nb = (b + NBUF - 1) % NBUF

                @pl.when(k + NBUF - 1 < NCHUNK)
                def _():
                    idx_wait(nb)
                    gather_issue(nb)

                @pl.when(k < NCHUNK)
                def _():
                    gather_wait(b)
                    scatter(b).wait()

                @pl.when(k + NBUF < NCHUNK)
                def _():
                    idx_issue(k + NBUF, b)
            return carry

        lax.fori_loop(0, NCHUNK // NBUF + 1, body, 0)

    def run_pair(ha, hb, oa, ob):
        stage(ha)
        plsc.subcore_barrier()
        run_quarter()
        plsc.subcore_barrier()
        readback(oa)
        stage(hb)
        plsc.subcore_barrier()
        run_quarter()
        plsc.subcore_barrier()
        readback(ob)

    @pl.when(c == 0)
    def _():
        run_pair(h0, h1, o0, o1)

    @pl.when(c == 1)
    def _():
        run_pair(h2, h3, o2, o3)


_BR = 2000


def _q_split(hw, q0_ref, q1_ref, q2_ref, q3_ref):
    q0_ref[...] = hw[:, 0 * QW:1 * QW]
    q1_ref[...] = hw[:, 1 * QW:2 * QW]
    q2_ref[...] = hw[:, 2 * QW:3 * QW]
    q3_ref[...] = hw[:, 3 * QW:4 * QW]


def _mm0_body(deg_ref, x_ref, w_ref, q0_ref, q1_ref, q2_ref, q3_ref,
              dinv_ref):
    deg = deg_ref[0, :, :1] + deg_ref[1, :, :1] - 1.0
    dinv = lax.rsqrt(deg)
    hw = jnp.dot(x_ref[...], w_ref[...], preferred_element_type=jnp.float32)
    hw = hw * dinv
    _q_split(hw, q0_ref, q1_ref, q2_ref, q3_ref)
    dinv_ref[...] = dinv


def _mm0_call(deg, x, w0):
    return pl.pallas_call(
        _mm0_body,
        grid=(N // _BR,),
        in_specs=[
            pl.BlockSpec((2, _BR, 16), lambda i: (0, i, 0)),
            pl.BlockSpec((_BR, D_IN), lambda i: (i, 0)),
            pl.BlockSpec((D_IN, H), lambda i: (0, 0)),
        ],
        out_specs=[pl.BlockSpec((_BR, QW), lambda i: (i, 0))] * 4
        + [pl.BlockSpec((_BR, 1), lambda i: (i, 0))],
        out_shape=[_QSD] * 4 + [jax.ShapeDtypeStruct((N, 1), jnp.float32)],
    )(deg, x, w0)


def _merge_body(q0_ref, q1_ref, q2_ref, q3_ref, dinv_ref, z_ref, s1_ref,
                s2_ref):
    i = pl.program_id(0)
    z = jnp.concatenate(
        [q0_ref[...], q1_ref[...], q2_ref[...], q3_ref[...]], axis=1)
    z = z * dinv_ref[...]
    z_ref[...] = z

    @pl.when(i == 0)
    def _():
        s1_ref[...] = jnp.zeros_like(s1_ref)
        s2_ref[...] = jnp.zeros_like(s2_ref)

    s1_ref[...] += jnp.sum(z, axis=0, keepdims=True)
    s2_ref[...] += jnp.sum(z * z, axis=0, keepdims=True)


def _merge_call(q0, q1, q2, q3, dinv):
    return pl.pallas_call(
        _merge_body,
        grid=(N // _BR,),
        in_specs=[pl.BlockSpec((_BR, QW), lambda i: (i, 0))] * 4
        + [pl.BlockSpec((_BR, 1), lambda i: (i, 0))],
        out_specs=[
            pl.BlockSpec((_BR, H), lambda i: (i, 0)),
            pl.BlockSpec((1, H), lambda i: (0, 0)),
            pl.BlockSpec((1, H), lambda i: (0, 0)),
        ],
        out_shape=[
            jax.ShapeDtypeStruct((N, H), jnp.float32),
            jax.ShapeDtypeStruct((1, H), jnp.float32),
            jax.ShapeDtypeStruct((1, H), jnp.float32),
        ],
    )(q0, q1, q2, q3, dinv)


def _bn_relu(z, s1, s2, g, be):
    mu = s1 * (1.0 / N)
    var = s2 * (1.0 / N) - mu * mu
    h = (z - mu) * (lax.rsqrt(var + 1e-5) * g) + be
    return jnp.maximum(h, 0.0)


def _norm_mm_body(z_ref, s1_ref, s2_ref, g_ref, be_ref, w_ref, dinv_ref,
                  q0_ref, q1_ref, q2_ref, q3_ref):
    h = _bn_relu(z_ref[...], s1_ref[...], s2_ref[...], g_ref[...], be_ref[...])
    hw = jnp.dot(h, w_ref[...], preferred_element_type=jnp.float32)
    hw = hw * dinv_ref[...]
    _q_split(hw, q0_ref, q1_ref, q2_ref, q3_ref)


def _norm_mm_call(z, s1, s2, g, be, w, dinv):
    return pl.pallas_call(
        _norm_mm_body,
        grid=(N // _BR,),
        in_specs=[
            pl.BlockSpec((_BR, H), lambda i: (i, 0)),
            pl.BlockSpec((1, H), lambda i: (0, 0)),
            pl.BlockSpec((1, H), lambda i: (0, 0)),
            pl.BlockSpec((1, H), lambda i: (0, 0)),
            pl.BlockSpec((1, H), lambda i: (0, 0)),
            pl.BlockSpec((H, H), lambda i: (0, 0)),
            pl.BlockSpec((_BR, 1), lambda i: (i, 0)),
        ],
        out_specs=[pl.BlockSpec((_BR, QW), lambda i: (i, 0))] * 4,
        out_shape=[_QSD] * 4,
    )(z, s1, s2, g, be, w, dinv)


def _head_body(z_ref, s1_ref, s2_ref, g_ref, be_ref, wf_ref, bf_ref, y_ref):
    h = _bn_relu(z_ref[...], s1_ref[...], s2_ref[...], g_ref[...], be_ref[...])
    y_ref[...] = jnp.dot(h, wf_ref[...],
                         preferred_element_type=jnp.float32) + bf_ref[...]


def _head_call(z, s1, s2, g, be, wf, bf):
    return pl.pallas_call(
        _head_body,
        grid=(N // _BR,),
        in_specs=[
            pl.BlockSpec((_BR, H), lambda i: (i, 0)),
            pl.BlockSpec((1, H), lambda i: (0, 0)),
            pl.BlockSpec((1, H), lambda i: (0, 0)),
            pl.BlockSpec((1, H), lambda i: (0, 0)),
            pl.BlockSpec((1, H), lambda i: (0, 0)),
            pl.BlockSpec((H, 1), lambda i: (0, 0)),
            pl.BlockSpec((1, 1), lambda i: (0, 0)),
        ],
        out_specs=pl.BlockSpec((_BR, 1), lambda i: (i, 0)),
        out_shape=jax.ShapeDtypeStruct((N, 1), jnp.float32),
    )(z, s1, s2, g, be, wf, bf)


def kernel(x, edge_index, W0, b0, g0, be0, W1, b1, g1, be1, W2, b2, g2, be2,
           Wf, bf):
    del b0, b1, b2
    src = edge_index[0]
    dst = edge_index[1]
    pad = EP - E
    srcp = jnp.concatenate([src, jnp.zeros((pad,), jnp.int32)])
    dstp = jnp.concatenate([dst, jnp.full((pad,), N, jnp.int32)])
    src3 = srcp.reshape(NSUB, NCHUNK, CHUNK)
    dst3 = dstp.reshape(NSUB, NCHUNK, CHUNK)
    sd4 = jnp.stack([src3, dst3], axis=2)
    ones16 = jnp.ones((CHUNK, 16), jnp.float32)

    deg2 = _deg_kernel(dst3, ones16)
    q0, q1, q2, q3, dinv = _mm0_call(deg2, x, W0)

    g0r, be0r = g0.reshape(1, H), be0.reshape(1, H)
    g1r, be1r = g1.reshape(1, H), be1.reshape(1, H)
    g2r, be2r = g2.reshape(1, H), be2.reshape(1, H)

    a0, a1, a2, a3 = _agg_kernel(q0, q1, q2, q3, sd4)
    z, s1, s2 = _merge_call(a0, a1, a2, a3, dinv)
    q0, q1, q2, q3 = _norm_mm_call(z, s1, s2, g0r, be0r, W1, dinv)

    a0, a1, a2, a3 = _agg_kernel(q0, q1, q2, q3, sd4)
    z, s1, s2 = _merge_call(a0, a1, a2, a3, dinv)
    q0, q1, q2, q3 = _norm_mm_call(z, s1, s2, g1r, be1r, W2, dinv)

    a0, a1, a2, a3 = _agg_kernel(q0, q1, q2, q3, sd4)
    z, s1, s2 = _merge_call(a0, a1, a2, a3, dinv)
    return _head_call(z, s1, s2, g2r, be2r, Wf, bf.reshape(1, 1))


# speedup vs baseline: 13.7397x; 1.3085x over previous
import functools

import jax
import jax.numpy as jnp
from jax import lax
from jax.experimental import pallas as pl
from jax.experimental.pallas import tpu as pltpu
from jax.experimental.pallas import tpu_sc as plsc

N = 10000
D_IN = 128
H = 256
QW = 64
E = 320000
EP = 327680
NSUB = 16
NE_TILE = EP // NSUB
CHUNK = 128
NCHUNK = NE_TILE // CHUNK
ROWS_TILE = 632
ROWS_LAST = N - 15 * ROWS_TILE
NACC = NSUB * ROWS_TILE
NBUF = 5
HCHUNK = NCHUNK // 2

_mesh = plsc.VectorSubcoreMesh(core_axis_name="c", subcore_axis_name="s")


@functools.partial(
    pl.kernel,
    mesh=_mesh,
    out_type=jax.ShapeDtypeStruct((2, N, 16), jnp.float32),
    scratch_types=[
        pltpu.VMEM((HCHUNK, CHUNK), jnp.int32),
        pltpu.VMEM((CHUNK, 16), jnp.float32),
        pltpu.VMEM_SHARED((NACC, 16), jnp.float32),
        pltpu.SemaphoreType.DMA,
        pltpu.SemaphoreType.DMA,
        pltpu.SemaphoreType.DMA,
        pltpu.SemaphoreType.DMA,
    ],
)
def _deg_kernel(dst_hbm, ones_hbm, deg_out, dslab, ones_v, acc_sh,
                s0, s1, s2, s3):
    c = lax.axis_index("c")
    s = lax.axis_index("s")
    sems = (s0, s1, s2, s3)

    pltpu.sync_copy(ones_hbm, ones_v)
    r0 = s * ROWS_TILE
    for j in range(4):
        pltpu.sync_copy(ones_v, acc_sh.at[pl.ds(r0 + j * CHUNK, CHUNK)])
    pltpu.sync_copy(ones_v.at[pl.ds(0, ROWS_TILE - 4 * CHUNK)],
                    acc_sh.at[pl.ds(r0 + 4 * CHUNK, ROWS_TILE - 4 * CHUNK)])
    pltpu.sync_copy(dst_hbm.at[s, pl.ds(c * HCHUNK, HCHUNK)], dslab)
    plsc.subcore_barrier()

    def body(g, carry):
        handles = []
        for b in range(4):
            k = g * 4 + b
            handles.append(pltpu.async_copy(
                ones_v, acc_sh.at[dslab.at[k]], sems[b], add=True))
        for h in handles:
            h.wait()
        return carry

    lax.fori_loop(0, HCHUNK // 4, body, 0)
    plsc.subcore_barrier()

    @pl.when(s < NSUB - 1)
    def _():
        pltpu.sync_copy(acc_sh.at[pl.ds(r0, ROWS_TILE)],
                        deg_out.at[c, pl.ds(r0, ROWS_TILE)])

    @pl.when(s == NSUB - 1)
    def _():
        pltpu.sync_copy(acc_sh.at[pl.ds(r0, ROWS_LAST)],
                        deg_out.at[c, pl.ds(r0, ROWS_LAST)])


_QSD = jax.ShapeDtypeStruct((N, QW), jnp.float32)


@functools.partial(
    pl.kernel,
    mesh=_mesh,
    compiler_params=pltpu.CompilerParams(use_tc_tiling_on_sc=False),
    out_type=[_QSD, _QSD, _QSD, _QSD],
    scratch_types=[
        pltpu.VMEM((HCHUNK, 2, CHUNK), jnp.int32),
        [pltpu.VMEM((CHUNK, QW), jnp.float32)] * 3,
        pltpu.VMEM_SHARED((NACC, QW), jnp.float32),
        pltpu.VMEM_SHARED((NACC, QW), jnp.float32),
        [pltpu.SemaphoreType.DMA] * 3,
        [pltpu.SemaphoreType.DMA] * 3,
    ],
)
def _agg_kernel(h0, h1, h2, h3, sd_hbm, o0, o1, o2, o3,
                slab, rowb, table_sh, acc_sh, gsems, ssems):
    c = lax.axis_index("c")
    s = lax.axis_index("s")
    r0 = s * ROWS_TILE

    def gather_issue(j, b):
        pltpu.async_copy(table_sh.at[slab.at[j, 0]], rowb[b], gsems[b])

    def gather_wait(b):
        pltpu.make_async_copy(table_sh.at[slab.at[0, 0]], rowb[b],
                              gsems[b]).wait()

    def scatter(j, b):
        return pltpu.async_copy(rowb[b], acc_sh.at[slab.at[j, 1]], ssems[b],
                                add=True)

    def stage(h):
        @pl.when(s < NSUB - 1)
        def _():
            pltpu.sync_copy(h.at[pl.ds(r0, ROWS_TILE)],
                            table_sh.at[pl.ds(r0, ROWS_TILE)])
            pltpu.sync_copy(h.at[pl.ds(r0, ROWS_TILE)],
                            acc_sh.at[pl.ds(r0, ROWS_TILE)])

        @pl.when(s == NSUB - 1)
        def _():
            pltpu.sync_copy(h.at[pl.ds(r0, ROWS_LAST)],
                            table_sh.at[pl.ds(r0, ROWS_LAST)])
            pltpu.sync_copy(h.at[pl.ds(r0, ROWS_LAST)],
                            acc_sh.at[pl.ds(r0, ROWS_LAST)])

    def readback(out):
        @pl.when(s < NSUB - 1)
        def _():
            pltpu.sync_copy(acc_sh.at[pl.ds(r0, ROWS_TILE)],
                            out.at[pl.ds(r0, ROWS_TILE)])

        @pl.when(s == NSUB - 1)
        def _():
            pltpu.sync_copy(acc_sh.at[pl.ds(r0, ROWS_LAST)],
                            out.at[pl.ds(r0, ROWS_LAST)])

    def run_quarter():
        for half in range(2):
            pltpu.sync_copy(sd_hbm.at[s, pl.ds(half * HCHUNK, HCHUNK)], slab)
            gather_issue(0, 0)
            gather_issue(1, 1)

            def body(g, carry):
                for b in range(3):
                    k = g * 3 + b

                    @pl.when(k + 2 < HCHUNK)
                    def _():
                        gather_issue(k + 2, (b + 2) % 3)

                    @pl.when(k < HCHUNK)
                    def _():
                        gather_wait(b)
                        scatter(k, b).wait()
                return carry

            lax.fori_loop(0, HCHUNK // 3 + 1, body, 0)

    def run_pair(ha, hb, oa, ob):
        stage(ha)
        plsc.subcore_barrier()
        run_quarter()
        plsc.subcore_barrier()
        readback(oa)
        stage(hb)
        plsc.subcore_barrier()
        run_quarter()
        plsc.subcore_barrier()
        readback(ob)

    @pl.when(c == 0)
    def _():
        run_pair(h0, h1, o0, o1)

    @pl.when(c == 1)
    def _():
        run_pair(h2, h3, o2, o3)


_BR = 2000


def _q_split(hw, q0_ref, q1_ref, q2_ref, q3_ref):
    q0_ref[...] = hw[:, 0 * QW:1 * QW]
    q1_ref[...] = hw[:, 1 * QW:2 * QW]
    q2_ref[...] = hw[:, 2 * QW:3 * QW]
    q3_ref[...] = hw[:, 3 * QW:4 * QW]


def _mm0_body(deg_ref, x_ref, w_ref, q0_ref, q1_ref, q2_ref, q3_ref,
              dinv_ref):
    deg = deg_ref[0, :, :1] + deg_ref[1, :, :1] - 1.0
    dinv = lax.rsqrt(deg)
    hw = jnp.dot(x_ref[...], w_ref[...], preferred_element_type=jnp.float32)
    hw = hw * dinv
    _q_split(hw, q0_ref, q1_ref, q2_ref, q3_ref)
    dinv_ref[...] = dinv


def _mm0_call(deg, x, w0):
    return pl.pallas_call(
        _mm0_body,
        grid=(N // _BR,),
        in_specs=[
            pl.BlockSpec((2, _BR, 16), lambda i: (0, i, 0)),
            pl.BlockSpec((_BR, D_IN), lambda i: (i, 0)),
            pl.BlockSpec((D_IN, H), lambda i: (0, 0)),
        ],
        out_specs=[pl.BlockSpec((_BR, QW), lambda i: (i, 0))] * 4
        + [pl.BlockSpec((_BR, 1), lambda i: (i, 0))],
        out_shape=[_QSD] * 4 + [jax.ShapeDtypeStruct((N, 1), jnp.float32)],
    )(deg, x, w0)


def _merge_body(q0_ref, q1_ref, q2_ref, q3_ref, dinv_ref, z_ref, s1_ref,
                s2_ref):
    i = pl.program_id(0)
    z = jnp.concatenate(
        [q0_ref[...], q1_ref[...], q2_ref[...], q3_ref[...]], axis=1)
    z = z * dinv_ref[...]
    z_ref[...] = z

    @pl.when(i == 0)
    def _():
        s1_ref[...] = jnp.zeros_like(s1_ref)
        s2_ref[...] = jnp.zeros_like(s2_ref)

    s1_ref[...] += jnp.sum(z, axis=0, keepdims=True)
    s2_ref[...] += jnp.sum(z * z, axis=0, keepdims=True)


def _merge_call(q0, q1, q2, q3, dinv):
    return pl.pallas_call(
        _merge_body,
        grid=(N // _BR,),
        in_specs=[pl.BlockSpec((_BR, QW), lambda i: (i, 0))] * 4
        + [pl.BlockSpec((_BR, 1), lambda i: (i, 0))],
        out_specs=[
            pl.BlockSpec((_BR, H), lambda i: (i, 0)),
            pl.BlockSpec((1, H), lambda i: (0, 0)),
            pl.BlockSpec((1, H), lambda i: (0, 0)),
        ],
        out_shape=[
            jax.ShapeDtypeStruct((N, H), jnp.float32),
            jax.ShapeDtypeStruct((1, H), jnp.float32),
            jax.ShapeDtypeStruct((1, H), jnp.float32),
        ],
    )(q0, q1, q2, q3, dinv)


def _bn_relu(z, s1, s2, g, be):
    mu = s1 * (1.0 / N)
    var = s2 * (1.0 / N) - mu * mu
    h = (z - mu) * (lax.rsqrt(var + 1e-5) * g) + be
    return jnp.maximum(h, 0.0)


def _norm_mm_body(z_ref, s1_ref, s2_ref, g_ref, be_ref, w_ref, dinv_ref,
                  q0_ref, q1_ref, q2_ref, q3_ref):
    h = _bn_relu(z_ref[...], s1_ref[...], s2_ref[...], g_ref[...], be_ref[...])
    hw = jnp.dot(h, w_ref[...], preferred_element_type=jnp.float32)
    hw = hw * dinv_ref[...]
    _q_split(hw, q0_ref, q1_ref, q2_ref, q3_ref)


def _norm_mm_call(z, s1, s2, g, be, w, dinv):
    return pl.pallas_call(
        _norm_mm_body,
        grid=(N // _BR,),
        in_specs=[
            pl.BlockSpec((_BR, H), lambda i: (i, 0)),
            pl.BlockSpec((1, H), lambda i: (0, 0)),
            pl.BlockSpec((1, H), lambda i: (0, 0)),
            pl.BlockSpec((1, H), lambda i: (0, 0)),
            pl.BlockSpec((1, H), lambda i: (0, 0)),
            pl.BlockSpec((H, H), lambda i: (0, 0)),
            pl.BlockSpec((_BR, 1), lambda i: (i, 0)),
        ],
        out_specs=[pl.BlockSpec((_BR, QW), lambda i: (i, 0))] * 4,
        out_shape=[_QSD] * 4,
    )(z, s1, s2, g, be, w, dinv)


def _head_body(z_ref, s1_ref, s2_ref, g_ref, be_ref, wf_ref, bf_ref, y_ref):
    h = _bn_relu(z_ref[...], s1_ref[...], s2_ref[...], g_ref[...], be_ref[...])
    y_ref[...] = jnp.dot(h, wf_ref[...],
                         preferred_element_type=jnp.float32) + bf_ref[...]


def _head_call(z, s1, s2, g, be, wf, bf):
    return pl.pallas_call(
        _head_body,
        grid=(N // _BR,),
        in_specs=[
            pl.BlockSpec((_BR, H), lambda i: (i, 0)),
            pl.BlockSpec((1, H), lambda i: (0, 0)),
            pl.BlockSpec((1, H), lambda i: (0, 0)),
            pl.BlockSpec((1, H), lambda i: (0, 0)),
            pl.BlockSpec((1, H), lambda i: (0, 0)),
            pl.BlockSpec((H, 1), lambda i: (0, 0)),
            pl.BlockSpec((1, 1), lambda i: (0, 0)),
        ],
        out_specs=pl.BlockSpec((_BR, 1), lambda i: (i, 0)),
        out_shape=jax.ShapeDtypeStruct((N, 1), jnp.float32),
    )(z, s1, s2, g, be, wf, bf)


def kernel(x, edge_index, W0, b0, g0, be0, W1, b1, g1, be1, W2, b2, g2, be2,
           Wf, bf):
    del b0, b1, b2
    src = edge_index[0]
    dst = edge_index[1]
    pad = EP - E
    srcp = jnp.concatenate([src, jnp.zeros((pad,), jnp.int32)])
    dstp = jnp.concatenate([dst, jnp.full((pad,), N, jnp.int32)])
    src3 = srcp.reshape(NSUB, NCHUNK, CHUNK)
    dst3 = dstp.reshape(NSUB, NCHUNK, CHUNK)
    sd4 = jnp.stack([src3, dst3], axis=2)
    ones16 = jnp.ones((CHUNK, 16), jnp.float32)

    deg2 = _deg_kernel(dst3, ones16)
    q0, q1, q2, q3, dinv = _mm0_call(deg2, x, W0)

    g0r, be0r = g0.reshape(1, H), be0.reshape(1, H)
    g1r, be1r = g1.reshape(1, H), be1.reshape(1, H)
    g2r, be2r = g2.reshape(1, H), be2.reshape(1, H)

    a0, a1, a2, a3 = _agg_kernel(q0, q1, q2, q3, sd4)
    z, s1, s2 = _merge_call(a0, a1, a2, a3, dinv)
    q0, q1, q2, q3 = _norm_mm_call(z, s1, s2, g0r, be0r, W1, dinv)

    a0, a1, a2, a3 = _agg_kernel(q0, q1, q2, q3, sd4)
    z, s1, s2 = _merge_call(a0, a1, a2, a3, dinv)
    q0, q1, q2, q3 = _norm_mm_call(z, s1, s2, g1r, be1r, W2, dinv)

    a0, a1, a2, a3 = _agg_kernel(q0, q1, q2, q3, sd4)
    z, s1, s2 = _merge_call(a0, a1, a2, a3, dinv)
    return _head_call(z, s1, s2, g2r, be2r, Wf, bf.reshape(1, 1))


# speedup vs baseline: 13.9772x; 1.0173x over previous
import functools

import jax
import jax.numpy as jnp
from jax import lax
from jax.experimental import pallas as pl
from jax.experimental.pallas import tpu as pltpu
from jax.experimental.pallas import tpu_sc as plsc

N = 10000
D_IN = 128
H = 256
QW = 64
E = 320000
EP = 327680
NSUB = 16
NE_TILE = EP // NSUB
CHUNK = 128
NCHUNK = NE_TILE // CHUNK
ROWS_TILE = 632
ROWS_LAST = N - 15 * ROWS_TILE
NACC = NSUB * ROWS_TILE
NBUF = 5
HCHUNK = NCHUNK // 2

_mesh = plsc.VectorSubcoreMesh(core_axis_name="c", subcore_axis_name="s")


@functools.partial(
    pl.kernel,
    mesh=_mesh,
    out_type=jax.ShapeDtypeStruct((2, N, 16), jnp.float32),
    scratch_types=[
        pltpu.VMEM((HCHUNK, CHUNK), jnp.int32),
        pltpu.VMEM((CHUNK, 16), jnp.float32),
        pltpu.VMEM_SHARED((NACC, 16), jnp.float32),
        pltpu.SemaphoreType.DMA,
        pltpu.SemaphoreType.DMA,
        pltpu.SemaphoreType.DMA,
        pltpu.SemaphoreType.DMA,
    ],
)
def _deg_kernel(dst_hbm, ones_hbm, deg_out, dslab, ones_v, acc_sh,
                s0, s1, s2, s3):
    c = lax.axis_index("c")
    s = lax.axis_index("s")
    sems = (s0, s1, s2, s3)

    pltpu.sync_copy(ones_hbm, ones_v)
    r0 = s * ROWS_TILE
    for j in range(4):
        pltpu.sync_copy(ones_v, acc_sh.at[pl.ds(r0 + j * CHUNK, CHUNK)])
    pltpu.sync_copy(ones_v.at[pl.ds(0, ROWS_TILE - 4 * CHUNK)],
                    acc_sh.at[pl.ds(r0 + 4 * CHUNK, ROWS_TILE - 4 * CHUNK)])
    pltpu.sync_copy(dst_hbm.at[s, pl.ds(c * HCHUNK, HCHUNK)], dslab)
    plsc.subcore_barrier()

    def body(g, carry):
        handles = []
        for b in range(4):
            k = g * 4 + b
            handles.append(pltpu.async_copy(
                ones_v, acc_sh.at[dslab.at[k]], sems[b], add=True))
        for h in handles:
            h.wait()
        return carry

    lax.fori_loop(0, HCHUNK // 4, body, 0)
    plsc.subcore_barrier()

    @pl.when(s < NSUB - 1)
    def _():
        pltpu.sync_copy(acc_sh.at[pl.ds(r0, ROWS_TILE)],
                        deg_out.at[c, pl.ds(r0, ROWS_TILE)])

    @pl.when(s == NSUB - 1)
    def _():
        pltpu.sync_copy(acc_sh.at[pl.ds(r0, ROWS_LAST)],
                        deg_out.at[c, pl.ds(r0, ROWS_LAST)])


_QSD = jax.ShapeDtypeStruct((N, QW), jnp.float32)


@functools.partial(
    pl.kernel,
    mesh=_mesh,
    compiler_params=pltpu.CompilerParams(use_tc_tiling_on_sc=False),
    out_type=[_QSD, _QSD, _QSD, _QSD],
    scratch_types=[
        pltpu.VMEM((HCHUNK, 2, CHUNK), jnp.int32),
        [pltpu.VMEM((CHUNK, QW), jnp.float32)] * 3,
        pltpu.VMEM_SHARED((NACC, QW), jnp.float32),
        pltpu.VMEM_SHARED((NACC, QW), jnp.float32),
        [pltpu.SemaphoreType.DMA] * 3,
        [pltpu.SemaphoreType.DMA] * 3,
    ],
)
def _agg_kernel(h0, h1, h2, h3, sd_hbm, o0, o1, o2, o3,
                slab, rowb, table_sh, acc_sh, gsems, ssems):
    c = lax.axis_index("c")
    s = lax.axis_index("s")
    r0 = s * ROWS_TILE

    def gather_issue(j, b):
        pltpu.async_copy(table_sh.at[slab.at[j, 0]], rowb[b], gsems[b])

    def gather_wait(b):
        pltpu.make_async_copy(table_sh.at[slab.at[0, 0]], rowb[b],
                              gsems[b]).wait()

    def scatter(j, b):
        return pltpu.async_copy(rowb[b], acc_sh.at[slab.at[j, 1]], ssems[b],
                                add=True)

    def stage(h):
        @pl.when(s < NSUB - 1)
        def _():
            pltpu.sync_copy(h.at[pl.ds(r0, ROWS_TILE)],
                            table_sh.at[pl.ds(r0, ROWS_TILE)])
            pltpu.sync_copy(h.at[pl.ds(r0, ROWS_TILE)],
                            acc_sh.at[pl.ds(r0, ROWS_TILE)])

        @pl.when(s == NSUB - 1)
        def _():
            pltpu.sync_copy(h.at[pl.ds(r0, ROWS_LAST)],
                            table_sh.at[pl.ds(r0, ROWS_LAST)])
            pltpu.sync_copy(h.at[pl.ds(r0, ROWS_LAST)],
                            acc_sh.at[pl.ds(r0, ROWS_LAST)])

    def readback(out):
        @pl.when(s < NSUB - 1)
        def _():
            pltpu.sync_copy(acc_sh.at[pl.ds(r0, ROWS_TILE)],
                            out.at[pl.ds(r0, ROWS_TILE)])

        @pl.when(s == NSUB - 1)
        def _():
            pltpu.sync_copy(acc_sh.at[pl.ds(r0, ROWS_LAST)],
                            out.at[pl.ds(r0, ROWS_LAST)])

    def run_quarter():
        for half in range(2):
            pltpu.sync_copy(sd_hbm.at[s, pl.ds(half * HCHUNK, HCHUNK)], slab)
            gather_issue(0, 0)
            gather_issue(1, 1)

            def body(g, carry):
                for b in range(3):
                    k = g * 3 + b

                    @pl.when(k + 2 < HCHUNK)
                    def _():
                        gather_issue(k + 2, (b + 2) % 3)

                    @pl.when(k < HCHUNK)
                    def _():
                        gather_wait(b)
                        scatter(k, b).wait()
                return carry

            lax.fori_loop(0, HCHUNK // 3 + 1, body, 0)

    def run_pair(ha, hb, oa, ob):
        stage(ha)
        plsc.subcore_barrier()
        run_quarter()
        plsc.subcore_barrier()
        readback(oa)
        stage(hb)
        plsc.subcore_barrier()
        run_quarter()
        plsc.subcore_barrier()
        readback(ob)

    @pl.when(c == 0)
    def _():
        run_pair(h0, h1, o0, o1)

    @pl.when(c == 1)
    def _():
        run_pair(h2, h3, o2, o3)


_BR = 2000


def _q_split(hw, q0_ref, q1_ref, q2_ref, q3_ref):
    q0_ref[...] = hw[:, 0 * QW:1 * QW]
    q1_ref[...] = hw[:, 1 * QW:2 * QW]
    q2_ref[...] = hw[:, 2 * QW:3 * QW]
    q3_ref[...] = hw[:, 3 * QW:4 * QW]


def _mm0_body(deg_ref, x_ref, w_ref, q0_ref, q1_ref, q2_ref, q3_ref,
              dinv_ref):
    deg = deg_ref[0, :, :1] + deg_ref[1, :, :1] - 1.0
    dinv = lax.rsqrt(deg)
    hw = jnp.dot(x_ref[...], w_ref[...], preferred_element_type=jnp.float32)
    hw = hw * dinv
    _q_split(hw, q0_ref, q1_ref, q2_ref, q3_ref)
    dinv_ref[...] = dinv


def _mm0_call(deg, x, w0):
    return pl.pallas_call(
        _mm0_body,
        grid=(N // _BR,),
        in_specs=[
            pl.BlockSpec((2, _BR, 16), lambda i: (0, i, 0)),
            pl.BlockSpec((_BR, D_IN), lambda i: (i, 0)),
            pl.BlockSpec((D_IN, H), lambda i: (0, 0)),
        ],
        out_specs=[pl.BlockSpec((_BR, QW), lambda i: (i, 0))] * 4
        + [pl.BlockSpec((_BR, 1), lambda i: (i, 0))],
        out_shape=[_QSD] * 4 + [jax.ShapeDtypeStruct((N, 1), jnp.float32)],
    )(deg, x, w0)


def _stats_phase(q0_ref, q1_ref, q2_ref, q3_ref, dinv_ref, z_scr,
                 s1_scr, s2_scr, i):
    z = jnp.concatenate(
        [q0_ref[...], q1_ref[...], q2_ref[...], q3_ref[...]], axis=1)
    z = z * dinv_ref[...]
    z_scr[pl.ds(i * _BR, _BR), :] = z

    @pl.when(i == 0)
    def _():
        s1_scr[...] = jnp.zeros_like(s1_scr)
        s2_scr[...] = jnp.zeros_like(s2_scr)

    s1_scr[...] += jnp.sum(z, axis=0, keepdims=True)
    s2_scr[...] += jnp.sum(z * z, axis=0, keepdims=True)


def _bn_relu(z, s1, s2, g, be):
    mu = s1 * (1.0 / N)
    var = s2 * (1.0 / N) - mu * mu
    h = (z - mu) * (lax.rsqrt(var + 1e-5) * g) + be
    return jnp.maximum(h, 0.0)


def _mnm_body(q0_ref, q1_ref, q2_ref, q3_ref, dinv_ref, g_ref, be_ref, w_ref,
              o0_ref, o1_ref, o2_ref, o3_ref, z_scr, s1_scr, s2_scr):
    p = pl.program_id(0)
    i = pl.program_id(1)

    @pl.when(p == 0)
    def _():
        _stats_phase(q0_ref, q1_ref, q2_ref, q3_ref, dinv_ref, z_scr,
                     s1_scr, s2_scr, i)

    @pl.when(p == 1)
    def _():
        z = z_scr[pl.ds(i * _BR, _BR), :]
        h = _bn_relu(z, s1_scr[...], s2_scr[...], g_ref[...], be_ref[...])
        hw = jnp.dot(h, w_ref[...], preferred_element_type=jnp.float32)
        hw = hw * dinv_ref[...]
        _q_split(hw, o0_ref, o1_ref, o2_ref, o3_ref)


def _mnm_call(q0, q1, q2, q3, dinv, g, be, w):
    return pl.pallas_call(
        _mnm_body,
        grid=(2, N // _BR),
        in_specs=[pl.BlockSpec((_BR, QW), lambda p, i: (i * (1 - p), 0))] * 4
        + [
            pl.BlockSpec((_BR, 1), lambda p, i: (i, 0)),
            pl.BlockSpec((1, H), lambda p, i: (0, 0)),
            pl.BlockSpec((1, H), lambda p, i: (0, 0)),
            pl.BlockSpec((H, H), lambda p, i: (0, 0)),
        ],
        out_specs=[pl.BlockSpec((_BR, QW), lambda p, i: (i * p, 0))] * 4,
        out_shape=[_QSD] * 4,
        scratch_shapes=[
            pltpu.VMEM((N, H), jnp.float32),
            pltpu.VMEM((1, H), jnp.float32),
            pltpu.VMEM((1, H), jnp.float32),
        ],
    )(q0, q1, q2, q3, dinv, g, be, w)


def _mh_body(q0_ref, q1_ref, q2_ref, q3_ref, dinv_ref, g_ref, be_ref, wf_ref,
             bf_ref, y_ref, z_scr, s1_scr, s2_scr):
    p = pl.program_id(0)
    i = pl.program_id(1)

    @pl.when(p == 0)
    def _():
        _stats_phase(q0_ref, q1_ref, q2_ref, q3_ref, dinv_ref, z_scr,
                     s1_scr, s2_scr, i)

    @pl.when(p == 1)
    def _():
        z = z_scr[pl.ds(i * _BR, _BR), :]
        h = _bn_relu(z, s1_scr[...], s2_scr[...], g_ref[...], be_ref[...])
        y_ref[...] = jnp.dot(h, wf_ref[...],
                             preferred_element_type=jnp.float32) + bf_ref[...]


def _mh_call(q0, q1, q2, q3, dinv, g, be, wf, bf):
    return pl.pallas_call(
        _mh_body,
        grid=(2, N // _BR),
        in_specs=[pl.BlockSpec((_BR, QW), lambda p, i: (i * (1 - p), 0))] * 4
        + [
            pl.BlockSpec((_BR, 1), lambda p, i: (i, 0)),
            pl.BlockSpec((1, H), lambda p, i: (0, 0)),
            pl.BlockSpec((1, H), lambda p, i: (0, 0)),
            pl.BlockSpec((H, 1), lambda p, i: (0, 0)),
            pl.BlockSpec((1, 1), lambda p, i: (0, 0)),
        ],
        out_specs=pl.BlockSpec((_BR, 1), lambda p, i: (i * p, 0)),
        out_shape=jax.ShapeDtypeStruct((N, 1), jnp.float32),
        scratch_shapes=[
            pltpu.VMEM((N, H), jnp.float32),
            pltpu.VMEM((1, H), jnp.float32),
            pltpu.VMEM((1, H), jnp.float32),
        ],
    )(q0, q1, q2, q3, dinv, g, be, wf, bf)


def kernel(x, edge_index, W0, b0, g0, be0, W1, b1, g1, be1, W2, b2, g2, be2,
           Wf, bf):
    del b0, b1, b2
    src = edge_index[0]
    dst = edge_index[1]
    pad = EP - E
    srcp = jnp.concatenate([src, jnp.zeros((pad,), jnp.int32)])
    dstp = jnp.concatenate([dst, jnp.full((pad,), N, jnp.int32)])
    src3 = srcp.reshape(NSUB, NCHUNK, CHUNK)
    dst3 = dstp.reshape(NSUB, NCHUNK, CHUNK)
    sd4 = jnp.stack([src3, dst3], axis=2)
    ones16 = jnp.ones((CHUNK, 16), jnp.float32)

    deg2 = _deg_kernel(dst3, ones16)
    q0, q1, q2, q3, dinv = _mm0_call(deg2, x, W0)

    g0r, be0r = g0.reshape(1, H), be0.reshape(1, H)
    g1r, be1r = g1.reshape(1, H), be1.reshape(1, H)
    g2r, be2r = g2.reshape(1, H), be2.reshape(1, H)

    a0, a1, a2, a3 = _agg_kernel(q0, q1, q2, q3, sd4)
    q0, q1, q2, q3 = _mnm_call(a0, a1, a2, a3, dinv, g0r, be0r, W1)

    a0, a1, a2, a3 = _agg_kernel(q0, q1, q2, q3, sd4)
    q0, q1, q2, q3 = _mnm_call(a0, a1, a2, a3, dinv, g1r, be1r, W2)

    a0, a1, a2, a3 = _agg_kernel(q0, q1, q2, q3, sd4)
    return _mh_call(a0, a1, a2, a3, dinv, g2r, be2r, Wf, bf.reshape(1, 1))
